# baseline (speedup 1.0000x reference)
import functools

import jax
import jax.numpy as jnp
from jax import lax
from jax.experimental import pallas as pl
from jax.experimental.pallas import tpu as pltpu
from jax.experimental.pallas import tpu_sc as plsc

B, F, V, D = 16384, 26, 1040000, 16
NW = 32
ROWS_W = B // NW
IDX_W = ROWS_W * F
CHUNK_ROWS = 16
CHUNK_IDX = CHUNK_ROWS * F
NCHUNKS = ROWS_W // CHUNK_ROWS

_mesh = plsc.VectorSubcoreMesh(core_axis_name="c", subcore_axis_name="s")

_TW = 8320


def _relayout_body(in_ref, out_ref):
    eye = jnp.eye(D, dtype=jnp.float32)
    out_ref[...] = lax.dot_general(
        in_ref[...], eye, (((0,), (0,)), ((), ())),
        preferred_element_type=jnp.float32)


def _relayout_emb2(emb2t):
    return pl.pallas_call(
        _relayout_body,
        grid=(V // _TW,),
        in_specs=[pl.BlockSpec((D, _TW), lambda s: (0, s))],
        out_specs=pl.BlockSpec((_TW, D), lambda s: (s, 0)),
        out_shape=jax.ShapeDtypeStruct((V, D), jnp.float32),
    )(emb2t)


@functools.partial(
    pl.kernel,
    mesh=_mesh,
    compiler_params=pltpu.CompilerParams(
        needs_layout_passes=False, use_tc_tiling_on_sc=False),
    out_type=jax.ShapeDtypeStruct((B,), jnp.float32),
    scratch_types=[
        pltpu.VMEM((NCHUNKS, CHUNK_IDX), jnp.int32),
        pltpu.VMEM((IDX_W,), jnp.int32),
        pltpu.VMEM((CHUNK_IDX, D), jnp.float32),
        pltpu.VMEM((IDX_W,), jnp.float32),
        pltpu.VMEM((ROWS_W,), jnp.float32),
        pltpu.VMEM((16,), jnp.float32),
        pltpu.VMEM((16,), jnp.float32),
        pltpu.SemaphoreType.DMA,
        pltpu.SemaphoreType.DMA,
    ],
)
def _fm_kernel(x_hbm, w_hbm, b_hbm, emb1_hbm, emb2_hbm, out_hbm,
               xv, xtv, e2b, e1all, outb, wv, bv, sem, sem1):
    wid = lax.axis_index("s") * 2 + lax.axis_index("c")
    pltpu.sync_copy(x_hbm.at[wid], xv)
    pltpu.sync_copy(w_hbm, wv)
    pltpu.sync_copy(b_hbm, bv)
    w = wv[...]
    bias = bv[...]
    lanes = lax.iota(jnp.int32, 16)
    base26 = lanes * F

    def transpose_body(c, carry):
        for f in range(F):
            vals = plsc.load_gather(xv, [jnp.full((16,), 0, jnp.int32) + c,
                                         base26 + f])
            xtv[pl.ds(c * CHUNK_IDX + f * CHUNK_ROWS, CHUNK_ROWS)] = vals
        return carry

    lax.fori_loop(0, NCHUNKS, transpose_body, 0)
    cp1 = pltpu.async_copy(emb1_hbm.at[xtv], e1all, sem1)

    def chunk_body(c, carry):
        pltpu.async_copy(emb2_hbm.at[xv.at[c]], e2b, sem).wait()

        sec = jnp.zeros((16,), jnp.float32)
        for r in range(CHUNK_ROWS):
            s = jnp.zeros((16,), jnp.float32)
            q = jnp.zeros((16,), jnp.float32)
            for f in range(F):
                v = e2b[r * F + f]
                s = s + v
                q = q + v * v
            sec = jnp.where(lanes == r, jnp.sum(s * s - q), sec)

        first = jnp.zeros((16,), jnp.float32)
        for f in range(F):
            first = first + e1all[pl.ds(c * CHUNK_IDX + f * CHUNK_ROWS,
                                        CHUNK_ROWS)]

        tot = (first + 0.5 * sec) * w + bias
        outb[pl.ds(c * CHUNK_ROWS, CHUNK_ROWS)] = 1.0 / (1.0 + jnp.exp(-tot))
        return carry

    cp1.wait()
    lax.fori_loop(0, NCHUNKS, chunk_body, 0)
    pltpu.sync_copy(outb, out_hbm.at[pl.ds(wid * ROWS_W, ROWS_W)])


def kernel(X, emb1, emb2, W, b):
    xr = X.astype(jnp.int32).reshape(NW, NCHUNKS, CHUNK_IDX)
    wvec = jnp.broadcast_to(W.astype(jnp.float32).reshape(1), (16,))
    bvec = jnp.broadcast_to(b.astype(jnp.float32).reshape(1), (16,))
    emb2r = _relayout_emb2(emb2.astype(jnp.float32).T)
    out = _fm_kernel(xr, wvec, bvec,
                     emb1.astype(jnp.float32).reshape(V),
                     emb2r)
    return out.reshape(B, 1)


# speedup vs baseline: 1.3140x; 1.3140x over previous
import functools

import jax
import jax.numpy as jnp
from jax import lax
from jax.experimental import pallas as pl
from jax.experimental.pallas import tpu as pltpu
from jax.experimental.pallas import tpu_sc as plsc

B, F, V, D = 16384, 26, 1040000, 16
NW = 32
ROWS_W = B // NW
IDX_W = ROWS_W * F
CHUNK_ROWS = 16
CHUNK_IDX = CHUNK_ROWS * F
NCHUNKS = ROWS_W // CHUNK_ROWS

_mesh = plsc.VectorSubcoreMesh(core_axis_name="c", subcore_axis_name="s")


@functools.partial(
    pl.kernel,
    mesh=_mesh,
    compiler_params=pltpu.CompilerParams(
        needs_layout_passes=False, use_tc_tiling_on_sc=False),
    out_type=jax.ShapeDtypeStruct((B,), jnp.float32),
    scratch_types=[
        pltpu.VMEM((NCHUNKS + 1, CHUNK_IDX), jnp.int32),
        pltpu.VMEM((IDX_W,), jnp.int32),
        pltpu.VMEM((CHUNK_IDX, D), jnp.float32),
        pltpu.VMEM((CHUNK_IDX, D), jnp.float32),
        pltpu.VMEM((IDX_W,), jnp.float32),
        pltpu.VMEM((ROWS_W,), jnp.float32),
        pltpu.VMEM((16,), jnp.float32),
        pltpu.VMEM((16,), jnp.float32),
        pltpu.SemaphoreType.DMA,
        pltpu.SemaphoreType.DMA,
        pltpu.SemaphoreType.DMA,
    ],
)
def _fm_kernel(x_hbm, w_hbm, b_hbm, emb1_hbm, emb2_hbm, out_hbm,
               xv, xtv, e2a, e2c, e1all, outb, wv, bv, semA, semB, sem1):
    wid = lax.axis_index("s") * 2 + lax.axis_index("c")
    pltpu.sync_copy(x_hbm.at[wid], xv.at[pl.ds(0, NCHUNKS)])
    pltpu.sync_copy(x_hbm.at[wid, 0], xv.at[NCHUNKS])
    pltpu.sync_copy(w_hbm, wv)
    pltpu.sync_copy(b_hbm, bv)
    w = wv[...]
    bias = bv[...]
    lanes = lax.iota(jnp.int32, 16)
    base26 = lanes * F

    def transpose_body(c, carry):
        for f in range(F):
            vals = plsc.load_gather(xv, [jnp.full((16,), 0, jnp.int32) + c,
                                         base26 + f])
            xtv[pl.ds(c * CHUNK_IDX + f * CHUNK_ROWS, CHUNK_ROWS)] = vals
        return carry

    lax.fori_loop(0, NCHUNKS, transpose_body, 0)
    cp1 = pltpu.async_copy(emb1_hbm.at[xtv], e1all, sem1)

    bufs = (e2a, e2c)
    sems = (semA, semB)
    pltpu.async_copy(emb2_hbm.at[xv.at[0]], e2a, semA)

    def compute_chunk(c, e2b):
        sec = jnp.zeros((16,), jnp.float32)
        for r in range(CHUNK_ROWS):
            s = jnp.zeros((16,), jnp.float32)
            q = jnp.zeros((16,), jnp.float32)
            for f in range(F):
                v = e2b[r * F + f]
                s = s + v
                q = q + v * v
            sec = jnp.where(lanes == r, jnp.sum(s * s - q), sec)

        first = jnp.zeros((16,), jnp.float32)
        for f in range(F):
            first = first + e1all[pl.ds(c * CHUNK_IDX + f * CHUNK_ROWS,
                                        CHUNK_ROWS)]

        tot = (first + 0.5 * sec) * w + bias
        outb[pl.ds(c * CHUNK_ROWS, CHUNK_ROWS)] = 1.0 / (1.0 + jnp.exp(-tot))

    def pair_body(p, carry):
        for k in range(2):
            c = 2 * p + k
            pltpu.async_copy(emb2_hbm.at[xv.at[c + 1]], bufs[1 - k],
                             sems[1 - k])
            pltpu.make_async_copy(emb2_hbm.at[xv.at[c]], bufs[k],
                                  sems[k]).wait()
            compute_chunk(c, bufs[k])
        return carry

    cp1.wait()
    lax.fori_loop(0, NCHUNKS // 2, pair_body, 0)
    pltpu.make_async_copy(emb2_hbm.at[xv.at[NCHUNKS]], e2a, semA).wait()
    pltpu.sync_copy(outb, out_hbm.at[pl.ds(wid * ROWS_W, ROWS_W)])


def kernel(X, emb1, emb2, W, b):
    xr = X.astype(jnp.int32).reshape(NW, NCHUNKS, CHUNK_IDX)
    wvec = jnp.broadcast_to(W.astype(jnp.float32).reshape(1), (16,))
    bvec = jnp.broadcast_to(b.astype(jnp.float32).reshape(1), (16,))
    out = _fm_kernel(xr, wvec, bvec,
                     emb1.astype(jnp.float32).reshape(V),
                     emb2.astype(jnp.float32))
    return out.reshape(B, 1)
